# Initial kernel scaffold; baseline (speedup 1.0000x reference)
import jax
import jax.numpy as jnp
from jax.experimental import pallas as pl


def kernel(group_idx, token_ids, all_weights):
    raise NotImplementedError("write your pallas kernel here")


# speedup vs baseline: 1.0049x; 1.0049x over previous
import functools

import jax
import jax.numpy as jnp
from jax import lax
from jax.experimental import pallas as pl
from jax.experimental.pallas import tpu as pltpu
from jax.experimental.pallas import tpu_sc as plsc

_VOCAB = 100000
_D = 64
_B = 4096 * 50
_NC = 2
_NS = 16
_NW = _NC * _NS
_BPW = _B // _NW
_CHUNK = 128
_NCHUNK = _BPW // _CHUNK
_L = 16


def _body(grp_hbm, ids_hbm, table_hbm, out_hbm, grp_v, idx_v, buf, sem):
    wid = lax.axis_index("s") * _NC + lax.axis_index("c")
    base = wid * _BPW

    pltpu.sync_copy(ids_hbm.at[pl.ds(base, _BPW)], idx_v)
    pltpu.sync_copy(grp_hbm, grp_v)
    off = grp_v[...] * _VOCAB

    def add_off(i, carry):
        sl = pl.ds(i * _L, _L)
        idx_v[sl] = idx_v[sl] + off
        return carry

    lax.fori_loop(0, _BPW // _L, add_off, 0)

    def chunk(c, carry):
        idx_sl = idx_v.at[pl.ds(c * _CHUNK, _CHUNK)]
        pltpu.async_copy(table_hbm.at[idx_sl], buf, sem).wait()
        pltpu.sync_copy(buf, out_hbm.at[pl.ds(base + c * _CHUNK, _CHUNK)])
        return carry

    lax.fori_loop(0, _NCHUNK, chunk, 0)


@jax.jit
def _run(grp16, ids_flat, table):
    mesh = plsc.VectorSubcoreMesh(core_axis_name="c", subcore_axis_name="s")
    f = pl.kernel(
        _body,
        mesh=mesh,
        out_type=jax.ShapeDtypeStruct((_B, _D), jnp.float32),
        scratch_types=[
            pltpu.VMEM((_L,), jnp.int32),
            pltpu.VMEM((_BPW,), jnp.int32),
            pltpu.VMEM((_CHUNK, _D), jnp.float32),
            pltpu.SemaphoreType.DMA,
        ],
        compiler_params=pltpu.CompilerParams(use_tc_tiling_on_sc=False),
    )
    return f(grp16, ids_flat, table)


def kernel(group_idx, token_ids, all_weights):
    grp16 = jnp.broadcast_to(group_idx.astype(jnp.int32), (_L,))
    ids_flat = token_ids.reshape(-1).astype(jnp.int32)
    out = _run(grp16, ids_flat, all_weights)
    return out.reshape(token_ids.shape[0], token_ids.shape[1], _D)


# speedup vs baseline: 1.0576x; 1.0525x over previous
import functools

import jax
import jax.numpy as jnp
from jax import lax
from jax.experimental import pallas as pl
from jax.experimental.pallas import tpu as pltpu
from jax.experimental.pallas import tpu_sc as plsc

_VOCAB = 100000
_D = 64
_B = 4096 * 50
_NC = 2
_NS = 16
_NW = _NC * _NS
_BPW = _B // _NW
_CHUNK = 128
_NBUF = 5
_NCHUNK = _BPW // _CHUNK
_NGROUP = _NCHUNK // _NBUF
_L = 16


def _body(grp_hbm, ids_hbm, table_hbm, out_hbm, grp_v, idx_v, bufs, gsem, wsem):
    wid = lax.axis_index("s") * _NC + lax.axis_index("c")
    base = wid * _BPW

    pltpu.sync_copy(ids_hbm.at[pl.ds(base, _BPW)], idx_v)
    pltpu.sync_copy(grp_hbm, grp_v)
    off = grp_v[...] * _VOCAB

    def add_off(i, carry):
        sl = pl.ds(i * _L, _L)
        idx_v[sl] = idx_v[sl] + off
        return carry

    lax.fori_loop(0, _BPW // _L, add_off, 0)

    def gather(c, b):
        idx_sl = idx_v.at[pl.ds(c * _CHUNK, _CHUNK)]
        pltpu.async_copy(table_hbm.at[idx_sl], bufs.at[b], gsem.at[b])

    def wait_gather(b):
        pltpu.make_async_copy(
            table_hbm.at[pl.ds(0, _CHUNK)], bufs.at[b], gsem.at[b]).wait()

    def write(c, b):
        pltpu.async_copy(
            bufs.at[b], out_hbm.at[pl.ds(base + c * _CHUNK, _CHUNK)], wsem.at[b])

    def wait_write(b):
        pltpu.make_async_copy(
            bufs.at[b], out_hbm.at[pl.ds(0, _CHUNK)], wsem.at[b]).wait()

    for b in range(_NBUF):
        gather(b, b)

    def group(g, carry):
        c0 = g * _NBUF
        for b in range(_NBUF):
            wait_gather(b)
            write(c0 + b, b)
        for b in range(_NBUF):
            wait_write(b)
            gather(c0 + _NBUF + b, b)
        return carry

    lax.fori_loop(0, _NGROUP - 1, group, 0)

    c0 = (_NGROUP - 1) * _NBUF
    for b in range(_NBUF):
        wait_gather(b)
        write(c0 + b, b)
    for b in range(_NBUF):
        wait_write(b)


@jax.jit
def _run(grp16, ids_flat, table):
    mesh = plsc.VectorSubcoreMesh(core_axis_name="c", subcore_axis_name="s")
    f = pl.kernel(
        _body,
        mesh=mesh,
        out_type=jax.ShapeDtypeStruct((_B, _D), jnp.float32),
        scratch_types=[
            pltpu.VMEM((_L,), jnp.int32),
            pltpu.VMEM((_BPW,), jnp.int32),
            pltpu.VMEM((_NBUF, _CHUNK, _D), jnp.float32),
            pltpu.SemaphoreType.DMA((_NBUF,)),
            pltpu.SemaphoreType.DMA((_NBUF,)),
        ],
        compiler_params=pltpu.CompilerParams(use_tc_tiling_on_sc=False),
    )
    return f(grp16, ids_flat, table)


def kernel(group_idx, token_ids, all_weights):
    grp16 = jnp.broadcast_to(group_idx.astype(jnp.int32), (_L,))
    ids_flat = token_ids.reshape(-1).astype(jnp.int32)
    out = _run(grp16, ids_flat, all_weights)
    return out.reshape(token_ids.shape[0], token_ids.shape[1], _D)


# speedup vs baseline: 1.3690x; 1.2944x over previous
import functools

import jax
import jax.numpy as jnp
from jax import lax
from jax.experimental import pallas as pl
from jax.experimental.pallas import tpu as pltpu
from jax.experimental.pallas import tpu_sc as plsc

_VOCAB = 100000
_D = 64
_BATCH = 4096
_SEQ = 50
_B = _BATCH * _SEQ
_NC = 2
_NS = 16
_NW = _NC * _NS
_BPW = _B // _NW
_TPB = 128
_L = 16


def _body(ids_hbm, tab_hbm, out_hbm, idx_v, idxs, buf, tbuf, gsem):
    wid = lax.axis_index("s") * _NC + lax.axis_index("c")
    pltpu.sync_copy(ids_hbm.at[pl.ds(wid * _BPW, _BPW)], idx_v)

    iota = lax.iota(jnp.int32, _L)

    def sblock(s, carry):
        for lg in range(8):
            tok = plsc.load_gather(idx_v, [(iota + _L * lg) * _SEQ + s])
            idxs[pl.ds(_L * lg, _L)] = tok
        pltpu.async_copy(tab_hbm.at[idxs], buf, gsem).wait()
        for d in range(_D):
            dcol = jnp.full((_L,), d, jnp.int32)
            for lg in range(8):
                v = plsc.load_gather(buf, [iota + _L * lg, dcol])
                tbuf[d // 8, d % 8, pl.ds(_L * lg, _L)] = v
        for i in range(8):
            pltpu.sync_copy(tbuf.at[i], out_hbm.at[s, i, wid])
        return carry

    lax.fori_loop(0, _SEQ, sblock, 0)


@jax.jit
def _run(ids_flat, tab):
    mesh = plsc.VectorSubcoreMesh(core_axis_name="c", subcore_axis_name="s")
    f = pl.kernel(
        _body,
        mesh=mesh,
        out_type=jax.ShapeDtypeStruct((_SEQ, 8, _NW, 8, 128), jnp.float32),
        scratch_types=[
            pltpu.VMEM((_BPW,), jnp.int32),
            pltpu.VMEM((_TPB,), jnp.int32),
            pltpu.VMEM((_TPB, _D), jnp.float32),
            pltpu.VMEM((8, 8, 128), jnp.float32),
            pltpu.SemaphoreType.DMA,
        ],
        compiler_params=pltpu.CompilerParams(
            use_tc_tiling_on_sc=False, needs_layout_passes=False),
    )
    return f(ids_flat, tab)


def kernel(group_idx, token_ids, all_weights):
    off = group_idx[0].astype(jnp.int32) * _VOCAB
    tab = lax.dynamic_slice(all_weights, (off, jnp.int32(0)), (_VOCAB, _D))
    ids_flat = token_ids.reshape(-1).astype(jnp.int32)
    out5 = _run(ids_flat, tab)
    return out5.transpose(2, 4, 0, 1, 3).reshape(_BATCH, _SEQ, _D)


# speedup vs baseline: 1.4814x; 1.0822x over previous
import functools

import jax
import jax.numpy as jnp
from jax import lax
from jax.experimental import pallas as pl
from jax.experimental.pallas import tpu as pltpu
from jax.experimental.pallas import tpu_sc as plsc

_VOCAB = 100000
_D = 64
_BATCH = 4096
_SEQ = 50
_B = _BATCH * _SEQ
_NC = 2
_NS = 16
_NW = _NC * _NS
_BPW = _B // _NW
_TPB = 128
_L = 16


def _body(ids_hbm, tab_hbm, out_hbm, idx_v, idxs, buf, tbuf, gsem, wsem):
    wid = lax.axis_index("s") * _NC + lax.axis_index("c")
    pltpu.sync_copy(ids_hbm.at[pl.ds(wid * _BPW, _BPW)], idx_v)

    iota = lax.iota(jnp.int32, _L)
    ngroup = _SEQ // 2

    def build_idx(s, b):
        for lg in range(8):
            tok = plsc.load_gather(idx_v, [(iota + _L * lg) * _SEQ + s])
            idxs[b, pl.ds(_L * lg, _L)] = tok

    def start_gather(b):
        pltpu.async_copy(tab_hbm.at[idxs.at[b]], buf.at[b], gsem.at[b])

    def wait_gather(b):
        pltpu.make_async_copy(
            tab_hbm.at[pl.ds(0, _TPB)], buf.at[b], gsem.at[b]).wait()

    def transpose(b):
        for d in range(_D):
            dcol = jnp.full((_L,), d, jnp.int32)
            for lg in range(8):
                v = plsc.load_gather(buf.at[b], [iota + _L * lg, dcol])
                tbuf[b, d // 8, d % 8, pl.ds(_L * lg, _L)] = v

    def start_writes(s, b):
        for i in range(8):
            pltpu.async_copy(tbuf.at[b, i], out_hbm.at[s, i, wid], wsem.at[b])

    def wait_writes(b):
        for i in range(8):
            pltpu.make_async_copy(
                tbuf.at[b, i], out_hbm.at[0, i, wid], wsem.at[b]).wait()

    for b in range(2):
        build_idx(b, b)
        start_gather(b)

    def group(g, carry):
        for b in range(2):
            s = 2 * g + b
            wait_gather(b)

            @pl.when(g < ngroup - 1)
            def _():
                build_idx(s + 2, b)

            @pl.when(g > 0)
            def _():
                wait_writes(b)

            transpose(b)
            start_writes(s, b)

            @pl.when(g < ngroup - 1)
            def _():
                start_gather(b)

        return carry

    lax.fori_loop(0, ngroup, group, 0)
    for b in range(2):
        wait_writes(b)


@jax.jit
def _run(ids_flat, tab):
    mesh = plsc.VectorSubcoreMesh(core_axis_name="c", subcore_axis_name="s")
    f = pl.kernel(
        _body,
        mesh=mesh,
        out_type=jax.ShapeDtypeStruct((_SEQ, 8, _NW, 8, 128), jnp.float32),
        scratch_types=[
            pltpu.VMEM((_BPW,), jnp.int32),
            pltpu.VMEM((2, _TPB), jnp.int32),
            pltpu.VMEM((2, _TPB, _D), jnp.float32),
            pltpu.VMEM((2, 8, 8, 128), jnp.float32),
            pltpu.SemaphoreType.DMA((2,)),
            pltpu.SemaphoreType.DMA((2,)),
        ],
        compiler_params=pltpu.CompilerParams(
            use_tc_tiling_on_sc=False, needs_layout_passes=False),
    )
    return f(ids_flat, tab)


def kernel(group_idx, token_ids, all_weights):
    off = group_idx[0].astype(jnp.int32) * _VOCAB
    tab = lax.dynamic_slice(all_weights, (off, jnp.int32(0)), (_VOCAB, _D))
    ids_flat = token_ids.reshape(-1).astype(jnp.int32)
    out5 = _run(ids_flat, tab)
    return out5.transpose(2, 4, 0, 1, 3).reshape(_BATCH, _SEQ, _D)


# speedup vs baseline: 1.8974x; 1.2808x over previous
import functools

import jax
import jax.numpy as jnp
from jax import lax
from jax.experimental import pallas as pl
from jax.experimental.pallas import tpu as pltpu
from jax.experimental.pallas import tpu_sc as plsc

_VOCAB = 100000
_D = 64
_BATCH = 4096
_SEQ = 50
_B = _BATCH * _SEQ
_NC = 2
_NS = 16
_NW = _NC * _NS
_TPB = 128
_L = 16


def _body(ids_hbm, tab_hbm, out_hbm, idx2, buf, tbuf, gsem, wsem):
    wid = lax.axis_index("s") * _NC + lax.axis_index("c")
    pltpu.sync_copy(ids_hbm.at[:, pl.ds(wid * _TPB, _TPB)], idx2)

    iota = lax.iota(jnp.int32, _L)
    rows = [iota + _L * lg for lg in range(8)]
    ones = jnp.ones((_L,), jnp.bool_)
    ngroup = _SEQ // 2

    def start_gather(s, b):
        pltpu.async_copy(tab_hbm.at[idx2.at[s]], buf.at[b], gsem.at[b])

    def wait_gather(b):
        pltpu.make_async_copy(
            tab_hbm.at[pl.ds(0, _TPB)], buf.at[b], gsem.at[b]).wait()

    def transpose(b):
        for d in range(_D):
            dcol = jnp.full((_L,), d, jnp.int32)
            vs = [plsc.load_gather(buf.at[b], [rows[lg], dcol])
                  for lg in range(8)]
            for lg in range(8):
                plsc.store_compressed(
                    tbuf.at[b, d // 8, d % 8, pl.ds(_L * lg, _L)],
                    vs[lg], mask=ones)

    def start_writes(s, b):
        for i in range(8):
            pltpu.async_copy(tbuf.at[b, i], out_hbm.at[s, i, wid], wsem.at[b])

    def wait_writes(b):
        for i in range(8):
            pltpu.make_async_copy(
                tbuf.at[b, i], out_hbm.at[0, i, wid], wsem.at[b]).wait()

    for b in range(2):
        start_gather(b, b)

    def group(g, carry):
        for b in range(2):
            s = 2 * g + b
            wait_gather(b)

            @pl.when(g > 0)
            def _():
                wait_writes(b)

            transpose(b)
            start_writes(s, b)

            @pl.when(g < ngroup - 1)
            def _():
                start_gather(s + 2, b)

        return carry

    lax.fori_loop(0, ngroup, group, 0)
    for b in range(2):
        wait_writes(b)


@jax.jit
def _run(ids2, tab):
    mesh = plsc.VectorSubcoreMesh(core_axis_name="c", subcore_axis_name="s")
    f = pl.kernel(
        _body,
        mesh=mesh,
        out_type=jax.ShapeDtypeStruct((_SEQ, 8, _NW, 8, 128), jnp.float32),
        scratch_types=[
            pltpu.VMEM((_SEQ, _TPB), jnp.int32),
            pltpu.VMEM((2, _TPB, _D), jnp.float32),
            pltpu.VMEM((2, 8, 8, 128), jnp.float32),
            pltpu.SemaphoreType.DMA((2,)),
            pltpu.SemaphoreType.DMA((2,)),
        ],
        compiler_params=pltpu.CompilerParams(
            use_tc_tiling_on_sc=False, needs_layout_passes=False),
    )
    return f(ids2, tab)


def kernel(group_idx, token_ids, all_weights):
    off = group_idx[0].astype(jnp.int32) * _VOCAB
    tab = lax.dynamic_slice(all_weights, (off, jnp.int32(0)), (_VOCAB, _D))
    ids2 = token_ids.T.astype(jnp.int32)
    out5 = _run(ids2, tab)
    return out5.transpose(2, 4, 0, 1, 3).reshape(_BATCH, _SEQ, _D)
